# baseline (speedup 1.0000x reference)
import functools

import jax
import jax.numpy as jnp
from jax.experimental import pallas as pl
from jax.experimental.pallas import tpu as pltpu

N_EMBD = 4096
N_EXPERTS = 64
TOP_K = 8

_TOKEN_BLOCK = 1024


def _topk_softmax(logits):
    tb = logits.shape[0]
    lane_f = jax.lax.broadcasted_iota(
        jnp.int32, (tb, N_EXPERTS), 1).astype(jnp.float32)
    big = jnp.float32(N_EXPERTS)
    cur = logits
    vals = []
    idxs_f = []
    for _ in range(TOP_K):
        m = jnp.max(cur, axis=1, keepdims=True)
        sel = jnp.min(jnp.where(cur == m, lane_f, big), axis=1, keepdims=True)
        vals.append(m)
        idxs_f.append(sel)
        cur = jnp.where(lane_f == sel, -jnp.inf, cur)
    top_vals = jnp.concatenate(vals, axis=1)
    top_idxs = jnp.concatenate(idxs_f, axis=1).astype(jnp.int32)

    e = jnp.exp(top_vals - top_vals[:, 0:1])
    return e / jnp.sum(e, axis=1, keepdims=True), top_idxs


def _router_block(x_ref, wt_ref, w_out_ref, i_out_ref, l_out_ref,
                  *, n_blocks):
    i = pl.program_id(0)

    @pl.when(i > 0)
    def _prev():
        base = (i - 1) * _TOKEN_BLOCK
        w, ix = _topk_softmax(l_out_ref[pl.ds(base, _TOKEN_BLOCK), :])
        w_out_ref[pl.ds(base, _TOKEN_BLOCK), :] = w
        i_out_ref[pl.ds(base, _TOKEN_BLOCK), :] = ix

    kh = N_EMBD // 2
    part0 = jax.lax.dot_general(
        x_ref[:, :kh], wt_ref[:kh, :],
        dimension_numbers=(((1,), (0,)), ((), ())),
        preferred_element_type=jnp.float32,
    )
    part1 = jax.lax.dot_general(
        x_ref[:, kh:], wt_ref[kh:, :],
        dimension_numbers=(((1,), (0,)), ((), ())),
        preferred_element_type=jnp.float32,
    )
    logits = part0 + part1
    l_out_ref[pl.ds(i * _TOKEN_BLOCK, _TOKEN_BLOCK), :] = logits

    @pl.when(i == n_blocks - 1)
    def _last():
        w, ix = _topk_softmax(logits)
        base = i * _TOKEN_BLOCK
        w_out_ref[pl.ds(base, _TOKEN_BLOCK), :] = w
        i_out_ref[pl.ds(base, _TOKEN_BLOCK), :] = ix


@functools.partial(jax.jit, static_argnames=("interpret",))
def kernel(x, W, interpret=False):
    b, t, c = x.shape
    n_tok = b * t
    xf = x.reshape(n_tok, c)
    wt = W.T
    n_blocks = n_tok // _TOKEN_BLOCK

    weights, indices, logits = pl.pallas_call(
        functools.partial(_router_block, n_blocks=n_blocks),
        grid=(n_blocks,),
        in_specs=[
            pl.BlockSpec((_TOKEN_BLOCK, c), lambda i: (i, 0)),
            pl.BlockSpec((c, N_EXPERTS), lambda i: (0, 0)),
        ],
        out_specs=[
            pl.BlockSpec((n_tok, TOP_K), lambda i: (0, 0)),
            pl.BlockSpec((n_tok, TOP_K), lambda i: (0, 0)),
            pl.BlockSpec((n_tok, N_EXPERTS), lambda i: (0, 0)),
        ],
        out_shape=[
            jax.ShapeDtypeStruct((n_tok, TOP_K), jnp.float32),
            jax.ShapeDtypeStruct((n_tok, TOP_K), jnp.int32),
            jax.ShapeDtypeStruct((n_tok, N_EXPERTS), jnp.float32),
        ],
        interpret=interpret,
    )(xf, wt)

    return (weights.reshape(b, t, TOP_K),
            indices.reshape(b, t, TOP_K),
            logits.reshape(b, t, N_EXPERTS))


# speedup vs baseline: 1.0048x; 1.0048x over previous
import functools

import jax
import jax.numpy as jnp
from jax.experimental import pallas as pl
from jax.experimental.pallas import tpu as pltpu

N_EMBD = 4096
N_EXPERTS = 64
TOP_K = 8

_TOKEN_BLOCK = 1024


def _topk_softmax(logits):
    tb = logits.shape[0]
    lane_f = jax.lax.broadcasted_iota(
        jnp.int32, (tb, N_EXPERTS), 1).astype(jnp.float32)
    big = jnp.float32(N_EXPERTS)
    cur = logits
    vals = []
    idxs_f = []
    for _ in range(TOP_K):
        m = jnp.max(cur, axis=1, keepdims=True)
        sel = jnp.min(jnp.where(cur == m, lane_f, big), axis=1, keepdims=True)
        vals.append(m)
        idxs_f.append(sel)
        cur = jnp.where(lane_f == sel, -jnp.inf, cur)
    top_vals = jnp.concatenate(vals, axis=1)
    top_idxs = jnp.concatenate(idxs_f, axis=1).astype(jnp.int32)

    e = jnp.exp(top_vals - top_vals[:, 0:1])
    return e / jnp.sum(e, axis=1, keepdims=True), top_idxs


def _router_block(x_ref, wt_ref, w_out_ref, i_out_ref, l_out_ref,
                  *, n_blocks):
    i = pl.program_id(0)

    @pl.when(i > 0)
    def _prev():
        base = (i - 1) * _TOKEN_BLOCK
        w, ix = _topk_softmax(l_out_ref[pl.ds(base, _TOKEN_BLOCK), :])
        w_out_ref[pl.ds(base, _TOKEN_BLOCK), :] = w
        i_out_ref[pl.ds(base, _TOKEN_BLOCK), :] = ix

    logits = jax.lax.dot_general(
        x_ref[...], wt_ref[...],
        dimension_numbers=(((1,), (0,)), ((), ())),
        preferred_element_type=jnp.float32,
    )
    l_out_ref[pl.ds(i * _TOKEN_BLOCK, _TOKEN_BLOCK), :] = logits

    @pl.when(i == n_blocks - 1)
    def _last():
        w, ix = _topk_softmax(logits)
        base = i * _TOKEN_BLOCK
        w_out_ref[pl.ds(base, _TOKEN_BLOCK), :] = w
        i_out_ref[pl.ds(base, _TOKEN_BLOCK), :] = ix


@functools.partial(jax.jit, static_argnames=("interpret",))
def kernel(x, W, interpret=False):
    b, t, c = x.shape
    n_tok = b * t
    xf = x.reshape(n_tok, c)
    wt = W.T
    n_blocks = n_tok // _TOKEN_BLOCK

    weights, indices, logits = pl.pallas_call(
        functools.partial(_router_block, n_blocks=n_blocks),
        grid=(n_blocks,),
        in_specs=[
            pl.BlockSpec((_TOKEN_BLOCK, c), lambda i: (i, 0)),
            pl.BlockSpec((c, N_EXPERTS), lambda i: (0, 0)),
        ],
        out_specs=[
            pl.BlockSpec((n_tok, TOP_K), lambda i: (0, 0)),
            pl.BlockSpec((n_tok, TOP_K), lambda i: (0, 0)),
            pl.BlockSpec((n_tok, N_EXPERTS), lambda i: (0, 0)),
        ],
        out_shape=[
            jax.ShapeDtypeStruct((n_tok, TOP_K), jnp.float32),
            jax.ShapeDtypeStruct((n_tok, TOP_K), jnp.int32),
            jax.ShapeDtypeStruct((n_tok, N_EXPERTS), jnp.float32),
        ],
        interpret=interpret,
    )(xf, wt)

    return (weights.reshape(b, t, TOP_K),
            indices.reshape(b, t, TOP_K),
            logits.reshape(b, t, N_EXPERTS))
